# baseline (speedup 1.0000x reference)
import functools

import jax
import jax.numpy as jnp
from jax import lax
from jax.experimental import pallas as pl
from jax.experimental.pallas import tpu as pltpu
from jax.experimental.pallas import tpu_sc as plsc

_B = 16384
_D = 32
_NC = 2
_NS = 16
_NW = _NC * _NS
_RPW = _B // _NW
_C = 256
_NCH = _RPW // _C
_CB = 512
_TOK = 4
_USER_V = 1000001
_TEXT_V = 10000
_OUT_W = 193
_AW = _OUT_W - _D

_UW = 65536
_UNB = 16
_USER_S = _UW * _UNB


def _repack_body(in_ref, *out_refs):
  for k in range(8):
    out_refs[k][...] = in_ref[k, :]


def _tc_repack(ut_t):
  return pl.pallas_call(
      _repack_body,
      grid=(_D // 8, _UNB),
      in_specs=[pl.BlockSpec((8, _UW), lambda g, j: (g, j))],
      out_specs=[pl.BlockSpec((_UW,), lambda g, j: (g * _UNB + j,))] * 8,
      out_shape=[jax.ShapeDtypeStruct(((_D // 8) * _USER_S,), jnp.float32)] * 8,
  )(ut_t)


def _sc_a_body(tsb_h, tsf_h, city_h, ctok_h, cat_h, gtok_h,
               ttab_h, ctab_h, cttab_h, gtab_h, gttab_h, par_h,
               out_h,
               tidx, cidx, gidx, tsf, ctokb, gtokb, ctcol, gtcol,
               crd, grd, tbuf, cbuf, gbuf, cacc, gacc,
               tsT, cT, gT, cteT, gteT, ntb, parv,
               sem_in, sem_g, sem_a, sem_w):
  wid = lax.axis_index("s") * _NC + lax.axis_index("c")
  lanes = lax.iota(jnp.int32, 16)

  for ch in range(_NCH):
    r0 = wid * _RPW + ch * _C

    stage = [
        pltpu.async_copy(tsb_h.at[pl.ds(r0, _C)], tidx, sem_in),
        pltpu.async_copy(city_h.at[pl.ds(r0, _C)], cidx, sem_in),
        pltpu.async_copy(cat_h.at[pl.ds(r0, _C)], gidx, sem_in),
        pltpu.async_copy(tsf_h.at[pl.ds(r0, _C)], tsf, sem_in),
    ]
    for t in range(_TOK):
      stage.append(pltpu.async_copy(
          ctok_h.at[pl.ds(t * _B + r0, _C)], ctokb.at[pl.ds(t * _C, _C)],
          sem_in))
      stage.append(pltpu.async_copy(
          gtok_h.at[pl.ds(t * _B + r0, _C)], gtokb.at[pl.ds(t * _C, _C)],
          sem_in))
    if ch == 0:
      stage.append(pltpu.async_copy(par_h, parv, sem_in))
    for cp in stage:
      cp.wait()

    sgath = [
        pltpu.async_copy(ttab_h.at[tidx], tbuf, sem_g),
        pltpu.async_copy(ctab_h.at[cidx], cbuf, sem_g),
        pltpu.async_copy(gtab_h.at[gidx], gbuf, sem_g),
    ]

    ones = jnp.full((16,), 1.0, jnp.float32)
    zf = jnp.zeros((16,), jnp.float32)
    zrow = jnp.full((16,), _TEXT_V, jnp.int32)

    def tok_group(g, carry):
      base = g * 16
      ccnt = zf
      gcnt = zf
      for t in range(_TOK):
        ct = ctokb[pl.ds(t * _C + base, 16)]
        gtk = gtokb[pl.ds(t * _C + base, 16)]
        cvalid = ct != 0
        gvalid = gtk != 0
        ccnt = ccnt + jnp.where(cvalid, ones, zf)
        gcnt = gcnt + jnp.where(gvalid, ones, zf)
        ctcol[pl.ds(t * _C + base, 16)] = jnp.where(cvalid, ct, zrow)
        gtcol[pl.ds(t * _C + base, 16)] = jnp.where(gvalid, gtk, zrow)
      crd[pl.ds(base, 16)] = ones / jnp.maximum(ccnt, ones)
      grd[pl.ds(base, 16)] = ones / jnp.maximum(gcnt, ones)
      return carry

    lax.fori_loop(0, _C // 16, tok_group, 0)

    c0 = pltpu.async_copy(cttab_h.at[ctcol.at[pl.ds(0, _C)]], cacc, sem_a)
    g0 = pltpu.async_copy(gttab_h.at[gtcol.at[pl.ds(0, _C)]], gacc, sem_a)
    c0.wait()
    g0.wait()
    adds = []
    for t in range(1, _TOK):
      adds.append(pltpu.async_copy(
          cttab_h.at[ctcol.at[pl.ds(t * _C, _C)]], cacc, sem_a, add=True))
      adds.append(pltpu.async_copy(
          gttab_h.at[gtcol.at[pl.ds(t * _C, _C)]], gacc, sem_a, add=True))

    mean = parv[pl.ds(0, 16)]
    istd = parv[pl.ds(16, 16)]

    def nt_group(g, carry):
      base = g * 16
      tv = tsf[pl.ds(base, 16)]
      ntb[0, pl.ds(base, 16)] = (tv - mean) * istd
      return carry

    lax.fori_loop(0, _C // 16, nt_group, 0)
    writes = [pltpu.async_copy(
        ntb, out_h.at[pl.ds(32, 1), pl.ds(r0, _C)], sem_w)]

    for gcp in sgath:
      gcp.wait()

    def tr_group(g, carry):
      base = g * 16
      rows = base + lanes
      for c in range(_D):
        csel = jnp.full((16,), c, jnp.int32)
        tsT[c, pl.ds(base, 16)] = plsc.load_gather(tbuf, [rows, csel])
        cT[c, pl.ds(base, 16)] = plsc.load_gather(cbuf, [rows, csel])
        gT[c, pl.ds(base, 16)] = plsc.load_gather(gbuf, [rows, csel])
      return carry

    lax.fori_loop(0, _C // 16, tr_group, 0)
    writes.append(pltpu.async_copy(
        tsT, out_h.at[pl.ds(0, _D), pl.ds(r0, _C)], sem_w))
    writes.append(pltpu.async_copy(
        cT, out_h.at[pl.ds(33, _D), pl.ds(r0, _C)], sem_w))
    writes.append(pltpu.async_copy(
        gT, out_h.at[pl.ds(97, _D), pl.ds(r0, _C)], sem_w))

    for a in adds:
      a.wait()

    def pool_group(g, carry):
      base = g * 16
      rows = base + lanes
      rc = crd[pl.ds(base, 16)]
      rg = grd[pl.ds(base, 16)]
      for c in range(_D):
        csel = jnp.full((16,), c, jnp.int32)
        cteT[c, pl.ds(base, 16)] = plsc.load_gather(cacc, [rows, csel]) * rc
        gteT[c, pl.ds(base, 16)] = plsc.load_gather(gacc, [rows, csel]) * rg
      return carry

    lax.fori_loop(0, _C // 16, pool_group, 0)
    writes.append(pltpu.async_copy(
        cteT, out_h.at[pl.ds(65, _D), pl.ds(r0, _C)], sem_w))
    writes.append(pltpu.async_copy(
        gteT, out_h.at[pl.ds(129, _D), pl.ds(r0, _C)], sem_w))

    for w in writes:
      w.wait()


def _sc_b_body(uid_h, u0_h, u1_h, u2_h, u3_h, u4_h, u5_h, u6_h, u7_h,
               out_h, uidx, uwidx, ucol, sem_in, sem_g, sem_w):
  wid = lax.axis_index("s") * _NC + lax.axis_index("c")
  r0 = wid * _CB
  utabs = (u0_h, u1_h, u2_h, u3_h, u4_h, u5_h, u6_h, u7_h)

  pltpu.async_copy(uid_h.at[pl.ds(r0, _CB)], uidx, sem_in).wait()

  def uw_group(g, carry):
    base = g * 16
    iv = uidx[pl.ds(base, 16)]
    for c in range(_D):
      uwidx[pl.ds(c * _CB + base, 16)] = iv + jnp.full(
          (16,), (c // 8) * _USER_S, jnp.int32)
    return carry

  lax.fori_loop(0, _CB // 16, uw_group, 0)

  ugath = []
  for c in range(_D):
    ugath.append(pltpu.async_copy(
        utabs[c % 8].at[uwidx.at[pl.ds(c * _CB, _CB)]], ucol.at[c], sem_g))
  for gcp in ugath:
    gcp.wait()

  pltpu.async_copy(ucol, out_h.at[:, pl.ds(r0, _CB)], sem_w).wait()


@functools.cache
def _sc_a():
  return functools.partial(
    pl.kernel,
    out_type=jax.ShapeDtypeStruct((_AW, _B), jnp.float32),
    mesh=plsc.VectorSubcoreMesh(
        core_axis_name="c", subcore_axis_name="s",
        num_cores=_NC, num_subcores=_NS),
    compiler_params=pltpu.CompilerParams(
        use_tc_tiling_on_sc=False, needs_layout_passes=False),
    scratch_types=[
        pltpu.VMEM((_C,), jnp.int32),
        pltpu.VMEM((_C,), jnp.int32),
        pltpu.VMEM((_C,), jnp.int32),
        pltpu.VMEM((_C,), jnp.float32),
        pltpu.VMEM((_TOK * _C,), jnp.int32),
        pltpu.VMEM((_TOK * _C,), jnp.int32),
        pltpu.VMEM((_TOK * _C,), jnp.int32),
        pltpu.VMEM((_TOK * _C,), jnp.int32),
        pltpu.VMEM((_C,), jnp.float32),
        pltpu.VMEM((_C,), jnp.float32),
        pltpu.VMEM((_C, _D), jnp.float32),
        pltpu.VMEM((_C, _D), jnp.float32),
        pltpu.VMEM((_C, _D), jnp.float32),
        pltpu.VMEM((_C, _D), jnp.float32),
        pltpu.VMEM((_C, _D), jnp.float32),
        pltpu.VMEM((_D, _C), jnp.float32),
        pltpu.VMEM((_D, _C), jnp.float32),
        pltpu.VMEM((_D, _C), jnp.float32),
        pltpu.VMEM((_D, _C), jnp.float32),
        pltpu.VMEM((_D, _C), jnp.float32),
        pltpu.VMEM((1, _C), jnp.float32),
        pltpu.VMEM((32,), jnp.float32),
        pltpu.SemaphoreType.DMA,
        pltpu.SemaphoreType.DMA,
        pltpu.SemaphoreType.DMA,
        pltpu.SemaphoreType.DMA,
    ],
  )(_sc_a_body)


@functools.cache
def _sc_b():
  return functools.partial(
    pl.kernel,
    out_type=jax.ShapeDtypeStruct((_D, _B), jnp.float32),
    mesh=plsc.VectorSubcoreMesh(
        core_axis_name="c", subcore_axis_name="s",
        num_cores=_NC, num_subcores=_NS),
    compiler_params=pltpu.CompilerParams(
        use_tc_tiling_on_sc=False, needs_layout_passes=False),
    scratch_types=[
        pltpu.VMEM((_CB,), jnp.int32),
        pltpu.VMEM((_D * _CB,), jnp.int32),
        pltpu.VMEM((_D, _CB), jnp.float32),
        pltpu.SemaphoreType.DMA,
        pltpu.SemaphoreType.DMA,
        pltpu.SemaphoreType.DMA,
    ],
  )(_sc_b_body)


def kernel(user_id, timestamp_bucket, timestamp, customer_city, city_tokens,
           product_category, cat_tokens, user_table, ts_table, city_table,
           city_text_table, cat_table, cat_text_table, norm_mean, norm_var):
  inv_std = lax.rsqrt(norm_var.astype(jnp.float32) + jnp.float32(1e-7))
  par = jnp.concatenate([
      jnp.full((16,), norm_mean, jnp.float32),
      jnp.full((16,), inv_std, jnp.float32),
  ])
  zero_row = jnp.zeros((1, _D), jnp.float32)
  ct_aug = jnp.concatenate([city_text_table, zero_row], axis=0)
  gt_aug = jnp.concatenate([cat_text_table, zero_row], axis=0)
  ctok_cm = city_tokens.T.reshape(-1)
  gtok_cm = cat_tokens.T.reshape(-1)
  rest = _sc_a()(
      timestamp_bucket, timestamp, customer_city, ctok_cm,
      product_category, gtok_cm, ts_table, city_table,
      ct_aug, cat_table, gt_aug, par)
  uts = _tc_repack(user_table.T)
  ublock = _sc_b()(user_id, *uts)
  return jnp.concatenate([ublock, rest], axis=0).T


# speedup vs baseline: 1.2286x; 1.2286x over previous
import functools

import jax
import jax.numpy as jnp
from jax import lax
from jax.experimental import pallas as pl
from jax.experimental.pallas import tpu as pltpu
from jax.experimental.pallas import tpu_sc as plsc

_B = 16384
_D = 32
_NC = 2
_NS = 16
_NW = _NC * _NS
_RPW = _B // _NW
_C = 256
_NCH = _RPW // _C
_CB = 512
_TOK = 4
_USER_V = 1000001
_TEXT_V = 10000
_OUT_W = 193
_AW = _OUT_W - _D

_UW = 65536
_UNB = 16
_USER_S = _UW * _UNB


def _repack_body(in_ref, *out_refs):
  for k in range(8):
    out_refs[k][...] = in_ref[k, :]


def _tc_repack(ut_t):
  return pl.pallas_call(
      _repack_body,
      grid=(_D // 8, _UNB),
      in_specs=[pl.BlockSpec((8, _UW), lambda g, j: (g, j))],
      out_specs=[pl.BlockSpec((_UW,), lambda g, j: (g * _UNB + j,))] * 8,
      out_shape=[jax.ShapeDtypeStruct(((_D // 8) * _USER_S,), jnp.float32)] * 8,
  )(ut_t)


def _sc_a_body(tsb_h, tsf_h, city_h, ctok_h, cat_h, gtok_h,
               ttab_h, ctab_h, cttab_h, gtab_h, gttab_h, par_h,
               out_h,
               tidx, cidx, gidx, tsf, ctokb, gtokb, ctcol, gtcol,
               crd, grd, tbuf, cbuf, gbuf, cacc, gacc,
               tsT, cT, gT, cteT, gteT, ntb, parv,
               sem_in, sem_g, sem_a, sem_w):
  wid = lax.axis_index("s") * _NC + lax.axis_index("c")
  lanes = lax.iota(jnp.int32, 16)

  for ch in range(_NCH):
    r0 = wid * _RPW + ch * _C

    stage = [
        pltpu.async_copy(tsb_h.at[pl.ds(r0, _C)], tidx, sem_in),
        pltpu.async_copy(city_h.at[pl.ds(r0, _C)], cidx, sem_in),
        pltpu.async_copy(cat_h.at[pl.ds(r0, _C)], gidx, sem_in),
        pltpu.async_copy(tsf_h.at[pl.ds(r0, _C)], tsf, sem_in),
    ]
    for t in range(_TOK):
      stage.append(pltpu.async_copy(
          ctok_h.at[pl.ds(t * _B + r0, _C)], ctokb.at[pl.ds(t * _C, _C)],
          sem_in))
      stage.append(pltpu.async_copy(
          gtok_h.at[pl.ds(t * _B + r0, _C)], gtokb.at[pl.ds(t * _C, _C)],
          sem_in))
    if ch == 0:
      stage.append(pltpu.async_copy(par_h, parv, sem_in))
    for cp in stage:
      cp.wait()

    sgath = [
        pltpu.async_copy(ttab_h.at[tidx], tbuf, sem_g),
        pltpu.async_copy(ctab_h.at[cidx], cbuf, sem_g),
        pltpu.async_copy(gtab_h.at[gidx], gbuf, sem_g),
    ]

    ones = jnp.full((16,), 1.0, jnp.float32)
    zf = jnp.zeros((16,), jnp.float32)
    zrow = jnp.full((16,), _TEXT_V, jnp.int32)

    def tok_group(g, carry):
      base = g * 16
      ccnt = zf
      gcnt = zf
      for t in range(_TOK):
        ct = ctokb[pl.ds(t * _C + base, 16)]
        gtk = gtokb[pl.ds(t * _C + base, 16)]
        cvalid = ct != 0
        gvalid = gtk != 0
        ccnt = ccnt + jnp.where(cvalid, ones, zf)
        gcnt = gcnt + jnp.where(gvalid, ones, zf)
        ctcol[pl.ds(t * _C + base, 16)] = jnp.where(cvalid, ct, zrow)
        gtcol[pl.ds(t * _C + base, 16)] = jnp.where(gvalid, gtk, zrow)
      crd[pl.ds(base, 16)] = ones / jnp.maximum(ccnt, ones)
      grd[pl.ds(base, 16)] = ones / jnp.maximum(gcnt, ones)
      return carry

    lax.fori_loop(0, _C // 16, tok_group, 0)

    c0 = pltpu.async_copy(cttab_h.at[ctcol.at[pl.ds(0, _C)]], cacc, sem_a)
    g0 = pltpu.async_copy(gttab_h.at[gtcol.at[pl.ds(0, _C)]], gacc, sem_a)
    c0.wait()
    g0.wait()
    adds = []
    for t in range(1, _TOK):
      adds.append(pltpu.async_copy(
          cttab_h.at[ctcol.at[pl.ds(t * _C, _C)]], cacc, sem_a, add=True))
      adds.append(pltpu.async_copy(
          gttab_h.at[gtcol.at[pl.ds(t * _C, _C)]], gacc, sem_a, add=True))

    mean = parv[pl.ds(0, 16)]
    istd = parv[pl.ds(16, 16)]

    def nt_group(g, carry):
      base = g * 16
      tv = tsf[pl.ds(base, 16)]
      ntb[0, pl.ds(base, 16)] = (tv - mean) * istd
      return carry

    lax.fori_loop(0, _C // 16, nt_group, 0)
    writes = [pltpu.async_copy(
        ntb, out_h.at[pl.ds(32, 1), pl.ds(r0, _C)], sem_w)]

    for gcp in sgath:
      gcp.wait()

    def tr_group(g, carry):
      base = g * 16
      rows = base + lanes
      for c in range(_D):
        csel = jnp.full((16,), c, jnp.int32)
        tsT[c, pl.ds(base, 16)] = plsc.load_gather(tbuf, [rows, csel])
        cT[c, pl.ds(base, 16)] = plsc.load_gather(cbuf, [rows, csel])
        gT[c, pl.ds(base, 16)] = plsc.load_gather(gbuf, [rows, csel])
      return carry

    lax.fori_loop(0, _C // 16, tr_group, 0)
    writes.append(pltpu.async_copy(
        tsT, out_h.at[pl.ds(0, _D), pl.ds(r0, _C)], sem_w))
    writes.append(pltpu.async_copy(
        cT, out_h.at[pl.ds(33, _D), pl.ds(r0, _C)], sem_w))
    writes.append(pltpu.async_copy(
        gT, out_h.at[pl.ds(97, _D), pl.ds(r0, _C)], sem_w))

    for a in adds:
      a.wait()

    def pool_group(g, carry):
      base = g * 16
      rows = base + lanes
      rc = crd[pl.ds(base, 16)]
      rg = grd[pl.ds(base, 16)]
      for c in range(_D):
        csel = jnp.full((16,), c, jnp.int32)
        cteT[c, pl.ds(base, 16)] = plsc.load_gather(cacc, [rows, csel]) * rc
        gteT[c, pl.ds(base, 16)] = plsc.load_gather(gacc, [rows, csel]) * rg
      return carry

    lax.fori_loop(0, _C // 16, pool_group, 0)
    writes.append(pltpu.async_copy(
        cteT, out_h.at[pl.ds(65, _D), pl.ds(r0, _C)], sem_w))
    writes.append(pltpu.async_copy(
        gteT, out_h.at[pl.ds(129, _D), pl.ds(r0, _C)], sem_w))

    for w in writes:
      w.wait()


def _sc_b_body(uid_h, u0_h, u1_h, u2_h, u3_h, u4_h, u5_h, u6_h, u7_h,
               out_h, uidx, uwidx, ucol, sem_in, sem_g, sem_w):
  wid = lax.axis_index("s") * _NC + lax.axis_index("c")
  r0 = wid * _CB
  utabs = (u0_h, u1_h, u2_h, u3_h, u4_h, u5_h, u6_h, u7_h)

  pltpu.async_copy(uid_h.at[pl.ds(r0, _CB)], uidx, sem_in).wait()

  def uw_group(g, carry):
    base = g * 16
    iv = uidx[pl.ds(base, 16)]
    for c in range(_D):
      uwidx[pl.ds(c * _CB + base, 16)] = iv + jnp.full(
          (16,), (c // 8) * _USER_S, jnp.int32)
    return carry

  lax.fori_loop(0, _CB // 16, uw_group, 0)

  ugath = []
  for c in range(_D):
    ugath.append(pltpu.async_copy(
        utabs[c % 8].at[uwidx.at[pl.ds(c * _CB, _CB)]], ucol.at[c], sem_g))
  for gcp in ugath:
    gcp.wait()

  pltpu.async_copy(ucol, out_h.at[:, pl.ds(r0, _CB)], sem_w).wait()


@functools.cache
def _sc_a():
  return functools.partial(
    pl.kernel,
    out_type=jax.ShapeDtypeStruct((_AW, _B), jnp.float32),
    mesh=plsc.VectorSubcoreMesh(
        core_axis_name="c", subcore_axis_name="s",
        num_cores=_NC, num_subcores=_NS),
    compiler_params=pltpu.CompilerParams(
        use_tc_tiling_on_sc=False, needs_layout_passes=False),
    scratch_types=[
        pltpu.VMEM((_C,), jnp.int32),
        pltpu.VMEM((_C,), jnp.int32),
        pltpu.VMEM((_C,), jnp.int32),
        pltpu.VMEM((_C,), jnp.float32),
        pltpu.VMEM((_TOK * _C,), jnp.int32),
        pltpu.VMEM((_TOK * _C,), jnp.int32),
        pltpu.VMEM((_TOK * _C,), jnp.int32),
        pltpu.VMEM((_TOK * _C,), jnp.int32),
        pltpu.VMEM((_C,), jnp.float32),
        pltpu.VMEM((_C,), jnp.float32),
        pltpu.VMEM((_C, _D), jnp.float32),
        pltpu.VMEM((_C, _D), jnp.float32),
        pltpu.VMEM((_C, _D), jnp.float32),
        pltpu.VMEM((_C, _D), jnp.float32),
        pltpu.VMEM((_C, _D), jnp.float32),
        pltpu.VMEM((_D, _C), jnp.float32),
        pltpu.VMEM((_D, _C), jnp.float32),
        pltpu.VMEM((_D, _C), jnp.float32),
        pltpu.VMEM((_D, _C), jnp.float32),
        pltpu.VMEM((_D, _C), jnp.float32),
        pltpu.VMEM((1, _C), jnp.float32),
        pltpu.VMEM((32,), jnp.float32),
        pltpu.SemaphoreType.DMA,
        pltpu.SemaphoreType.DMA,
        pltpu.SemaphoreType.DMA,
        pltpu.SemaphoreType.DMA,
    ],
  )(_sc_a_body)


@functools.cache
def _sc_b():
  return functools.partial(
    pl.kernel,
    out_type=jax.ShapeDtypeStruct((_D, _B), jnp.float32),
    mesh=plsc.VectorSubcoreMesh(
        core_axis_name="c", subcore_axis_name="s",
        num_cores=_NC, num_subcores=_NS),
    compiler_params=pltpu.CompilerParams(
        use_tc_tiling_on_sc=False, needs_layout_passes=False),
    scratch_types=[
        pltpu.VMEM((_CB,), jnp.int32),
        pltpu.VMEM((_D * _CB,), jnp.int32),
        pltpu.VMEM((_D, _CB), jnp.float32),
        pltpu.SemaphoreType.DMA,
        pltpu.SemaphoreType.DMA,
        pltpu.SemaphoreType.DMA,
    ],
  )(_sc_b_body)


def kernel(user_id, timestamp_bucket, timestamp, customer_city, city_tokens,
           product_category, cat_tokens, user_table, ts_table, city_table,
           city_text_table, cat_table, cat_text_table, norm_mean, norm_var):
  inv_std = lax.rsqrt(norm_var.astype(jnp.float32) + jnp.float32(1e-7))
  par = jnp.concatenate([
      jnp.full((16,), norm_mean, jnp.float32),
      jnp.full((16,), inv_std, jnp.float32),
  ])
  zero_row = jnp.zeros((1, _D), jnp.float32)
  ct_aug = jnp.concatenate([city_text_table, zero_row], axis=0)
  gt_aug = jnp.concatenate([cat_text_table, zero_row], axis=0)
  ctok_cm = city_tokens.T.reshape(-1)
  gtok_cm = cat_tokens.T.reshape(-1)
  rest = _sc_a()(
      timestamp_bucket, timestamp, customer_city, ctok_cm,
      product_category, gtok_cm, ts_table, city_table,
      ct_aug, cat_table, gt_aug, par)
  uts = _tc_repack(user_table.T)
  user_id_b, rest = lax.optimization_barrier((user_id, rest))
  ublock = _sc_b()(user_id_b, *uts)
  return jnp.concatenate([ublock, rest], axis=0).T


# speedup vs baseline: 1.2299x; 1.0010x over previous
import functools

import jax
import jax.numpy as jnp
from jax import lax
from jax.experimental import pallas as pl
from jax.experimental.pallas import tpu as pltpu
from jax.experimental.pallas import tpu_sc as plsc

_B = 16384
_D = 32
_NC = 2
_NS = 16
_NW = _NC * _NS
_RPW = _B // _NW
_C = 256
_NCH = _RPW // _C
_CB = 512
_TOK = 4
_USER_V = 1000001
_TEXT_V = 10000
_OUT_W = 193
_AW = _OUT_W - _D

_UW = 65536
_UNB = 16
_USER_S = _UW * _UNB


def _repack_body(in_ref, *out_refs):
  for k in range(8):
    out_refs[k][...] = in_ref[k, :]


def _tc_repack(ut_t):
  return pl.pallas_call(
      _repack_body,
      grid=(_D // 8, _UNB),
      in_specs=[pl.BlockSpec((8, _UW), lambda g, j: (g, j))],
      out_specs=[pl.BlockSpec((_UW,), lambda g, j: (g * _UNB + j,))] * 8,
      out_shape=[jax.ShapeDtypeStruct(((_D // 8) * _USER_S,), jnp.float32)] * 8,
  )(ut_t)


def _sc_a_body(tsb_h, tsf_h, city_h, ctok_h, cat_h, gtok_h,
               ttab_h, ctab_h, cttab_h, gtab_h, gttab_h, par_h,
               out_h,
               tidx, cidx, gidx, tsf, ctokb, gtokb, ctcol, gtcol,
               crd, grd, tbuf, cbuf, gbuf, cacc, gacc,
               tsT, cT, gT, cteT, gteT, ntb, parv,
               sem_in, sem_g, sem_a, sem_w):
  wid = lax.axis_index("s") * _NC + lax.axis_index("c")
  lanes = lax.iota(jnp.int32, 16)

  for ch in range(_NCH):
    r0 = wid * _RPW + ch * _C

    stage = [
        pltpu.async_copy(tsb_h.at[pl.ds(r0, _C)], tidx, sem_in),
        pltpu.async_copy(city_h.at[pl.ds(r0, _C)], cidx, sem_in),
        pltpu.async_copy(cat_h.at[pl.ds(r0, _C)], gidx, sem_in),
        pltpu.async_copy(tsf_h.at[pl.ds(r0, _C)], tsf, sem_in),
    ]
    for t in range(_TOK):
      stage.append(pltpu.async_copy(
          ctok_h.at[pl.ds(t * _B + r0, _C)], ctokb.at[pl.ds(t * _C, _C)],
          sem_in))
      stage.append(pltpu.async_copy(
          gtok_h.at[pl.ds(t * _B + r0, _C)], gtokb.at[pl.ds(t * _C, _C)],
          sem_in))
    if ch == 0:
      stage.append(pltpu.async_copy(par_h, parv, sem_in))
    for cp in stage:
      cp.wait()

    sgath = [
        pltpu.async_copy(ttab_h.at[tidx], tbuf, sem_g),
        pltpu.async_copy(ctab_h.at[cidx], cbuf, sem_g),
        pltpu.async_copy(gtab_h.at[gidx], gbuf, sem_g),
    ]

    ones = jnp.full((16,), 1.0, jnp.float32)
    zf = jnp.zeros((16,), jnp.float32)
    zrow = jnp.full((16,), _TEXT_V, jnp.int32)

    def tok_group(g, carry):
      base = g * 16
      rows = base + lanes
      for c in range(_D):
        csel = jnp.full((16,), c, jnp.int32)
        plsc.store_scatter(cacc, [rows, csel], zf)
        plsc.store_scatter(gacc, [rows, csel], zf)
      ccnt = zf
      gcnt = zf
      for t in range(_TOK):
        ct = ctokb[pl.ds(t * _C + base, 16)]
        gtk = gtokb[pl.ds(t * _C + base, 16)]
        cvalid = ct != 0
        gvalid = gtk != 0
        ccnt = ccnt + jnp.where(cvalid, ones, zf)
        gcnt = gcnt + jnp.where(gvalid, ones, zf)
        ctcol[pl.ds(t * _C + base, 16)] = jnp.where(cvalid, ct, zrow)
        gtcol[pl.ds(t * _C + base, 16)] = jnp.where(gvalid, gtk, zrow)
      crd[pl.ds(base, 16)] = ones / jnp.maximum(ccnt, ones)
      grd[pl.ds(base, 16)] = ones / jnp.maximum(gcnt, ones)
      return carry

    lax.fori_loop(0, _C // 16, tok_group, 0)

    adds = []
    for t in range(_TOK):
      adds.append(pltpu.async_copy(
          cttab_h.at[ctcol.at[pl.ds(t * _C, _C)]], cacc, sem_a, add=True))
      adds.append(pltpu.async_copy(
          gttab_h.at[gtcol.at[pl.ds(t * _C, _C)]], gacc, sem_a, add=True))

    mean = parv[pl.ds(0, 16)]
    istd = parv[pl.ds(16, 16)]

    def nt_group(g, carry):
      base = g * 16
      tv = tsf[pl.ds(base, 16)]
      ntb[0, pl.ds(base, 16)] = (tv - mean) * istd
      return carry

    lax.fori_loop(0, _C // 16, nt_group, 0)
    writes = [pltpu.async_copy(
        ntb, out_h.at[pl.ds(32, 1), pl.ds(r0, _C)], sem_w)]

    for gcp in sgath:
      gcp.wait()

    def tr_group(g, carry):
      base = g * 16
      rows = base + lanes
      for c in range(_D):
        csel = jnp.full((16,), c, jnp.int32)
        tsT[c, pl.ds(base, 16)] = plsc.load_gather(tbuf, [rows, csel])
        cT[c, pl.ds(base, 16)] = plsc.load_gather(cbuf, [rows, csel])
        gT[c, pl.ds(base, 16)] = plsc.load_gather(gbuf, [rows, csel])
      return carry

    lax.fori_loop(0, _C // 16, tr_group, 0)
    writes.append(pltpu.async_copy(
        tsT, out_h.at[pl.ds(0, _D), pl.ds(r0, _C)], sem_w))
    writes.append(pltpu.async_copy(
        cT, out_h.at[pl.ds(33, _D), pl.ds(r0, _C)], sem_w))
    writes.append(pltpu.async_copy(
        gT, out_h.at[pl.ds(97, _D), pl.ds(r0, _C)], sem_w))

    for a in adds:
      a.wait()

    def pool_group(g, carry):
      base = g * 16
      rows = base + lanes
      rc = crd[pl.ds(base, 16)]
      rg = grd[pl.ds(base, 16)]
      for c in range(_D):
        csel = jnp.full((16,), c, jnp.int32)
        cteT[c, pl.ds(base, 16)] = plsc.load_gather(cacc, [rows, csel]) * rc
        gteT[c, pl.ds(base, 16)] = plsc.load_gather(gacc, [rows, csel]) * rg
      return carry

    lax.fori_loop(0, _C // 16, pool_group, 0)
    writes.append(pltpu.async_copy(
        cteT, out_h.at[pl.ds(65, _D), pl.ds(r0, _C)], sem_w))
    writes.append(pltpu.async_copy(
        gteT, out_h.at[pl.ds(129, _D), pl.ds(r0, _C)], sem_w))

    for w in writes:
      w.wait()


def _sc_b_body(uid_h, u0_h, u1_h, u2_h, u3_h, u4_h, u5_h, u6_h, u7_h,
               out_h, uidx, uwidx, ucol, sem_in, sem_g, sem_w):
  wid = lax.axis_index("s") * _NC + lax.axis_index("c")
  r0 = wid * _CB
  utabs = (u0_h, u1_h, u2_h, u3_h, u4_h, u5_h, u6_h, u7_h)

  pltpu.async_copy(uid_h.at[pl.ds(r0, _CB)], uidx, sem_in).wait()

  def uw_group(g, carry):
    base = g * 16
    iv = uidx[pl.ds(base, 16)]
    for c in range(_D):
      uwidx[pl.ds(c * _CB + base, 16)] = iv + jnp.full(
          (16,), (c // 8) * _USER_S, jnp.int32)
    return carry

  lax.fori_loop(0, _CB // 16, uw_group, 0)

  ugath = []
  for c in range(_D):
    ugath.append(pltpu.async_copy(
        utabs[c % 8].at[uwidx.at[pl.ds(c * _CB, _CB)]], ucol.at[c], sem_g))
  for gcp in ugath:
    gcp.wait()

  pltpu.async_copy(ucol, out_h.at[:, pl.ds(r0, _CB)], sem_w).wait()


@functools.cache
def _sc_a():
  return functools.partial(
    pl.kernel,
    out_type=jax.ShapeDtypeStruct((_AW, _B), jnp.float32),
    mesh=plsc.VectorSubcoreMesh(
        core_axis_name="c", subcore_axis_name="s",
        num_cores=_NC, num_subcores=_NS),
    compiler_params=pltpu.CompilerParams(
        use_tc_tiling_on_sc=False, needs_layout_passes=False),
    scratch_types=[
        pltpu.VMEM((_C,), jnp.int32),
        pltpu.VMEM((_C,), jnp.int32),
        pltpu.VMEM((_C,), jnp.int32),
        pltpu.VMEM((_C,), jnp.float32),
        pltpu.VMEM((_TOK * _C,), jnp.int32),
        pltpu.VMEM((_TOK * _C,), jnp.int32),
        pltpu.VMEM((_TOK * _C,), jnp.int32),
        pltpu.VMEM((_TOK * _C,), jnp.int32),
        pltpu.VMEM((_C,), jnp.float32),
        pltpu.VMEM((_C,), jnp.float32),
        pltpu.VMEM((_C, _D), jnp.float32),
        pltpu.VMEM((_C, _D), jnp.float32),
        pltpu.VMEM((_C, _D), jnp.float32),
        pltpu.VMEM((_C, _D), jnp.float32),
        pltpu.VMEM((_C, _D), jnp.float32),
        pltpu.VMEM((_D, _C), jnp.float32),
        pltpu.VMEM((_D, _C), jnp.float32),
        pltpu.VMEM((_D, _C), jnp.float32),
        pltpu.VMEM((_D, _C), jnp.float32),
        pltpu.VMEM((_D, _C), jnp.float32),
        pltpu.VMEM((1, _C), jnp.float32),
        pltpu.VMEM((32,), jnp.float32),
        pltpu.SemaphoreType.DMA,
        pltpu.SemaphoreType.DMA,
        pltpu.SemaphoreType.DMA,
        pltpu.SemaphoreType.DMA,
    ],
  )(_sc_a_body)


@functools.cache
def _sc_b():
  return functools.partial(
    pl.kernel,
    out_type=jax.ShapeDtypeStruct((_D, _B), jnp.float32),
    mesh=plsc.VectorSubcoreMesh(
        core_axis_name="c", subcore_axis_name="s",
        num_cores=_NC, num_subcores=_NS),
    compiler_params=pltpu.CompilerParams(
        use_tc_tiling_on_sc=False, needs_layout_passes=False),
    scratch_types=[
        pltpu.VMEM((_CB,), jnp.int32),
        pltpu.VMEM((_D * _CB,), jnp.int32),
        pltpu.VMEM((_D, _CB), jnp.float32),
        pltpu.SemaphoreType.DMA,
        pltpu.SemaphoreType.DMA,
        pltpu.SemaphoreType.DMA,
    ],
  )(_sc_b_body)


def kernel(user_id, timestamp_bucket, timestamp, customer_city, city_tokens,
           product_category, cat_tokens, user_table, ts_table, city_table,
           city_text_table, cat_table, cat_text_table, norm_mean, norm_var):
  inv_std = lax.rsqrt(norm_var.astype(jnp.float32) + jnp.float32(1e-7))
  par = jnp.concatenate([
      jnp.full((16,), norm_mean, jnp.float32),
      jnp.full((16,), inv_std, jnp.float32),
  ])
  zero_row = jnp.zeros((1, _D), jnp.float32)
  ct_aug = jnp.concatenate([city_text_table, zero_row], axis=0)
  gt_aug = jnp.concatenate([cat_text_table, zero_row], axis=0)
  ctok_cm = city_tokens.T.reshape(-1)
  gtok_cm = cat_tokens.T.reshape(-1)
  rest = _sc_a()(
      timestamp_bucket, timestamp, customer_city, ctok_cm,
      product_category, gtok_cm, ts_table, city_table,
      ct_aug, cat_table, gt_aug, par)
  uts = _tc_repack(user_table.T)
  user_id_b, rest = lax.optimization_barrier((user_id, rest))
  ublock = _sc_b()(user_id_b, *uts)
  return jnp.concatenate([ublock, rest], axis=0).T
